# baseline (speedup 1.0000x reference)
import functools
import jax
import jax.numpy as jnp
from jax import lax
from jax.experimental import pallas as pl
from jax.experimental.pallas import tpu as pltpu
from jax.experimental.pallas import tpu_sc as plsc

B, P, D = 4, 2048, 1024
NW = 32
P_W = P // NW
SUB = 16
N_SUB = P_W // SUB
CHUNK = SUB * D


def _sc_body(x_hbm, lut_hbm, out_hbm, lut_v, xa_v, xb_v, sem_in, sem_out):
    wid = lax.axis_index("s") * 2 + lax.axis_index("c")
    base = wid * P_W * D
    pltpu.sync_copy(lut_hbm.at[pl.ds(base, P_W * D)], lut_v)

    bufs = (xa_v, xb_v)

    def unit(b, s, buf):
        off = base + s * CHUNK
        pltpu.async_copy(x_hbm.at[b, pl.ds(off, CHUNK)], buf, sem_in).wait()

        def add_one(i, _):
            o = i * 16
            buf[pl.ds(o, 16)] = buf[pl.ds(o, 16)] + lut_v[pl.ds(s * CHUNK + o, 16)]
            return 0

        lax.fori_loop(0, CHUNK // 16, add_one, 0)
        pltpu.async_copy(buf, out_hbm.at[b, pl.ds(off, CHUNK)], sem_out).wait()

    for b in range(B):
        for s in range(N_SUB):
            unit(b, s, bufs[(b * N_SUB + s) % 2])


def kernel(x, lut_weight):
    mesh = plsc.VectorSubcoreMesh(core_axis_name="c", subcore_axis_name="s")
    f = functools.partial(
        pl.kernel,
        mesh=mesh,
        out_type=jax.ShapeDtypeStruct((B, P * D), jnp.float32),
        scratch_types=[
            pltpu.VMEM((P_W * D,), jnp.float32),
            pltpu.VMEM((CHUNK,), jnp.float32),
            pltpu.VMEM((CHUNK,), jnp.float32),
            pltpu.SemaphoreType.DMA,
            pltpu.SemaphoreType.DMA,
        ],
    )(_sc_body)
    out = f(x.reshape(B, P * D), lut_weight.reshape(P * D))
    return out.reshape(B, P, D)


# speedup vs baseline: 1.9825x; 1.9825x over previous
import functools
import jax
import jax.numpy as jnp
from jax import lax
from jax.experimental import pallas as pl
from jax.experimental.pallas import tpu as pltpu
from jax.experimental.pallas import tpu_sc as plsc

B, P, D = 4, 2048, 1024
NW = 32
NBUF = 4
LOOK = 2


def _make_sc_body(P_W, SUB, N_SUB):
    U = B * N_SUB
    NG = U // NBUF

    def body(x_hbm, lut_hbm, out_hbm, lut_v, b0, b1, b2, b3,
             si0, si1, si2, si3, so0, so1, so2, so3):
        bufs = (b0, b1, b2, b3)
        sin = (si0, si1, si2, si3)
        sout = (so0, so1, so2, so3)
        wid = lax.axis_index("c") * 16 + lax.axis_index("s")
        p0 = wid * P_W
        pltpu.sync_copy(lut_hbm.at[pl.ds(p0, P_W)], lut_v)

        def unit_pos(k):
            b = k // N_SUB
            s = k % N_SUB
            return b, p0 + s * SUB, s * SUB

        def start_in(k, i):
            b, row, _ = unit_pos(k)
            pltpu.async_copy(x_hbm.at[b, pl.ds(row, SUB)], bufs[i], sin[i])

        def wait_in(i):
            pltpu.make_async_copy(
                x_hbm.at[0, pl.ds(0, SUB)], bufs[i], sin[i]).wait()

        def start_out(k, i):
            b, row, _ = unit_pos(k)
            pltpu.async_copy(bufs[i], out_hbm.at[b, pl.ds(row, SUB)], sout[i])

        def wait_out(i):
            pltpu.make_async_copy(
                bufs[i], out_hbm.at[0, pl.ds(0, SUB)], sout[i]).wait()

        for k in range(LOOK):
            start_in(k, k % NBUF)

        def group(g, _):
            for i in range(NBUF):
                k = g * NBUF + i
                inext = (i + LOOK) % NBUF
                @pl.when(k + LOOK < U)
                def _():
                    @pl.when(k + LOOK >= NBUF)
                    def _():
                        wait_out(inext)
                    start_in(k + LOOK, inext)

                wait_in(i)
                _, _, lrow = unit_pos(k)

                def add_row(r, _):
                    for c in range(D // 16):
                        o = c * 16
                        bufs[i][r, pl.ds(o, 16)] = (
                            bufs[i][r, pl.ds(o, 16)]
                            + lut_v[lrow + r, pl.ds(o, 16)])
                    return 0

                lax.fori_loop(0, SUB, add_row, 0)
                start_out(k, i)
            return 0

        lax.fori_loop(0, NG, group, 0)
        for i in range(min(NBUF, U)):
            wait_out(i)

    return body


def kernel(x, lut_weight):
    P_W = P // NW
    SUB = 8
    N_SUB = P_W // SUB
    mesh = plsc.VectorSubcoreMesh(core_axis_name="c", subcore_axis_name="s")
    f = functools.partial(
        pl.kernel,
        mesh=mesh,
        out_type=jax.ShapeDtypeStruct((B, P, D), jnp.float32),
        scratch_types=(
            [pltpu.VMEM((P_W, D), jnp.float32)]
            + [pltpu.VMEM((SUB, D), jnp.float32)] * NBUF
            + [pltpu.SemaphoreType.DMA] * (2 * NBUF)
        ),
    )(_make_sc_body(P_W, SUB, N_SUB))
    return f(x, lut_weight)


# speedup vs baseline: 2.6511x; 1.3373x over previous
import functools
import jax
import jax.numpy as jnp
from jax import lax
from jax.experimental import pallas as pl
from jax.experimental.pallas import tpu as pltpu
from jax.experimental.pallas import tpu_sc as plsc

B, P, D = 4, 2048, 1024
NW = 32
NBUF = 4
LOOK = 2
B_TC = 3


def _make_sc_body(P_W, SUB, N_SUB, B_LO, NB):
    U = NB * N_SUB
    NG = U // NBUF

    def body(x_hbm, lut_hbm, out_hbm, lut_v, b0, b1, b2, b3,
             si0, si1, si2, si3, so0, so1, so2, so3):
        bufs = (b0, b1, b2, b3)
        sin = (si0, si1, si2, si3)
        sout = (so0, so1, so2, so3)
        wid = lax.axis_index("c") * 16 + lax.axis_index("s")
        p0 = wid * P_W
        pltpu.sync_copy(lut_hbm.at[pl.ds(p0, P_W)], lut_v)

        def unit_pos(k):
            b = B_LO + k // N_SUB
            s = k % N_SUB
            return b, p0 + s * SUB, s * SUB

        def start_in(k, i):
            b, row, _ = unit_pos(k)
            pltpu.async_copy(x_hbm.at[b, pl.ds(row, SUB)], bufs[i], sin[i])

        def wait_in(i):
            pltpu.make_async_copy(
                x_hbm.at[0, pl.ds(0, SUB)], bufs[i], sin[i]).wait()

        def start_out(k, i):
            b, row, _ = unit_pos(k)
            pltpu.async_copy(
                bufs[i], out_hbm.at[b - B_LO, pl.ds(row, SUB)], sout[i])

        def wait_out(i):
            pltpu.make_async_copy(
                bufs[i], out_hbm.at[0, pl.ds(0, SUB)], sout[i]).wait()

        for k in range(LOOK):
            start_in(k, k % NBUF)

        def group(g, _):
            for i in range(NBUF):
                k = g * NBUF + i
                inext = (i + LOOK) % NBUF
                @pl.when(k + LOOK < U)
                def _():
                    @pl.when(k + LOOK >= NBUF)
                    def _():
                        wait_out(inext)
                    start_in(k + LOOK, inext)

                wait_in(i)
                _, _, lrow = unit_pos(k)

                def add_row(r, _):
                    for c in range(D // 16):
                        o = c * 16
                        bufs[i][r, pl.ds(o, 16)] = (
                            bufs[i][r, pl.ds(o, 16)]
                            + lut_v[lrow + r, pl.ds(o, 16)])
                    return 0

                lax.fori_loop(0, SUB, add_row, 0)
                start_out(k, i)
            return 0

        lax.fori_loop(0, NG, group, 0)
        for i in range(min(NBUF, U)):
            wait_out(i)

    return body


def _sc_part(x, lut_weight, b_lo, nb):
    P_W = P // NW
    SUB = 8
    N_SUB = P_W // SUB
    mesh = plsc.VectorSubcoreMesh(core_axis_name="c", subcore_axis_name="s")
    f = functools.partial(
        pl.kernel,
        mesh=mesh,
        out_type=jax.ShapeDtypeStruct((nb, P, D), jnp.float32),
        scratch_types=(
            [pltpu.VMEM((P_W, D), jnp.float32)]
            + [pltpu.VMEM((SUB, D), jnp.float32)] * NBUF
            + [pltpu.SemaphoreType.DMA] * (2 * NBUF)
        ),
    )(_make_sc_body(P_W, SUB, N_SUB, b_lo, nb))
    return f(x, lut_weight)


def _tc_add_body(x_ref, lut_ref, o_ref):
    o_ref[...] = x_ref[...] + lut_ref[...]


def _tc_part(x, lut_weight, nb):
    return pl.pallas_call(
        _tc_add_body,
        grid=(1, nb),
        in_specs=[
            pl.BlockSpec((1, P, D), lambda i, j: (j, i, 0)),
            pl.BlockSpec((P, D), lambda i, j: (i, 0)),
        ],
        out_specs=pl.BlockSpec((1, P, D), lambda i, j: (j, i, 0)),
        out_shape=jax.ShapeDtypeStruct((nb, P, D), x.dtype),
    )(x, lut_weight)


def kernel(x, lut_weight):
    tc_out = _tc_part(x, lut_weight, B_TC)
    sc_out = _sc_part(x, lut_weight, B_TC, B - B_TC)
    return jnp.concatenate([tc_out, sc_out], axis=0)


# speedup vs baseline: 8.9941x; 3.3925x over previous
import jax
import jax.numpy as jnp
from jax.experimental import pallas as pl
from jax.experimental.pallas import tpu as pltpu

BLK_P = 2048


def _add_body(x_ref, lut_ref, o_ref):
    o_ref[...] = x_ref[...] + lut_ref[...]


def kernel(x, lut_weight):
    B, P, D = x.shape
    grid = (P // BLK_P, B)
    return pl.pallas_call(
        _add_body,
        grid=grid,
        in_specs=[
            pl.BlockSpec((1, BLK_P, D), lambda i, j: (j, i, 0)),
            pl.BlockSpec((BLK_P, D), lambda i, j: (i, 0)),
        ],
        out_specs=pl.BlockSpec((1, BLK_P, D), lambda i, j: (j, i, 0)),
        out_shape=jax.ShapeDtypeStruct((B, P, D), x.dtype),
    )(x, lut_weight)
